# baseline (device time: 43281 ns/iter reference)
import jax
import jax.numpy as jnp
from jax import lax
from jax.experimental import pallas as pl
from jax.experimental.pallas import tpu as pltpu

N_DEV = 8
B, SQ, SKV, HQ, DH = 2, 128, 128, 32, 64
H_LOC = HQ // N_DEV
D_MODEL = 512


def kernel(x, Wq, K_ext, V_ext, Wo):
    my = lax.axis_index("i")
    k_loc = lax.dynamic_slice_in_dim(K_ext, my * H_LOC, H_LOC, axis=2)
    v_loc = lax.dynamic_slice_in_dim(V_ext, my * H_LOC, H_LOC, axis=2)
    k_loc = k_loc.transpose(0, 2, 1, 3).reshape(B * H_LOC, SKV, DH)
    v_loc = v_loc.transpose(0, 2, 1, 3).reshape(B * H_LOC, SKV, DH)

    def body(x_ref, wq_ref, k_ref, v_ref, wo_ref, out_ref,
             comm_ref, send_sems, recv_sems):
        my = lax.axis_index("i")

        barrier = pltpu.get_barrier_semaphore()
        for delta in range(1, N_DEV):
            pl.semaphore_signal(
                barrier, inc=1,
                device_id=((my + delta) % N_DEV,),
                device_id_type=pl.DeviceIdType.MESH,
            )
        pl.semaphore_wait(barrier, N_DEV - 1)

        wq = wq_ref[...]
        wo = wo_ref[...]
        parts = []
        for b in range(B):
            q_b = jnp.dot(x_ref[b], wq, preferred_element_type=jnp.float32)
            ctx = []
            for h in range(H_LOC):
                i = b * H_LOC + h
                q_bh = q_b[:, h * DH:(h + 1) * DH]
                s = lax.dot_general(
                    q_bh, k_ref[i], (((1,), (1,)), ((), ())),
                    preferred_element_type=jnp.float32,
                ) * 0.125
                m = jnp.max(s, axis=-1, keepdims=True)
                w = jnp.exp(s - m)
                w = w / jnp.sum(w, axis=-1, keepdims=True)
                ctx.append(jnp.dot(w, v_ref[i],
                                   preferred_element_type=jnp.float32))
            ctx_b = jnp.concatenate(ctx, axis=1)
            parts.append(jnp.dot(ctx_b, wo,
                                 preferred_element_type=jnp.float32))
        partial = jnp.concatenate(parts, axis=0)
        comm_ref[0] = partial

        rdmas = []
        for delta in range(1, N_DEV):
            r = pltpu.make_async_remote_copy(
                src_ref=comm_ref.at[0],
                dst_ref=comm_ref.at[delta],
                send_sem=send_sems.at[delta],
                recv_sem=recv_sems.at[delta],
                device_id=((my + delta) % N_DEV,),
                device_id_type=pl.DeviceIdType.MESH,
            )
            r.start()
            rdmas.append(r)

        acc = partial
        for delta in range(1, N_DEV):
            rdmas[delta - 1].wait_recv()
            acc = acc + comm_ref[delta]
        for r in rdmas:
            r.wait_send()
        out_ref[...] = acc.reshape(B, SQ, D_MODEL)

    return pl.pallas_call(
        body,
        out_shape=jax.ShapeDtypeStruct((B, SQ, D_MODEL), jnp.float32),
        in_specs=[pl.BlockSpec(memory_space=pltpu.VMEM)] * 5,
        out_specs=pl.BlockSpec(memory_space=pltpu.VMEM),
        scratch_shapes=[
            pltpu.VMEM((N_DEV, B * SQ, D_MODEL), jnp.float32),
            pltpu.SemaphoreType.DMA((N_DEV,)),
            pltpu.SemaphoreType.DMA((N_DEV,)),
        ],
        compiler_params=pltpu.CompilerParams(collective_id=0),
    )(x, Wq, k_loc, v_loc, Wo)


# device time: 20662 ns/iter; 2.0947x vs baseline; 2.0947x over previous
import jax
import jax.numpy as jnp
from jax import lax
from jax.experimental import pallas as pl
from jax.experimental.pallas import tpu as pltpu

N_DEV = 8
B, SQ, SKV, HQ, DH = 2, 128, 128, 32, 64
H_LOC = HQ // N_DEV
D_MODEL = 512
ROWS = B * SQ
R8 = ROWS // N_DEV


def kernel(x, Wq, K_ext, V_ext, Wo):
    my = lax.axis_index("i")
    k_loc = lax.dynamic_slice_in_dim(K_ext, my * H_LOC, H_LOC, axis=2)
    v_loc = lax.dynamic_slice_in_dim(V_ext, my * H_LOC, H_LOC, axis=2)
    k_loc = k_loc.transpose(0, 2, 1, 3).reshape(B * H_LOC, SKV, DH)
    v_loc = v_loc.transpose(0, 2, 1, 3).reshape(B * H_LOC, SKV, DH)

    def body(x_ref, wq_ref, k_ref, v_ref, wo_ref, out_ref,
             part_ref, rs_ref, ag_ref,
             rs_send, rs_recv, ag_send, ag_recv):
        my = lax.axis_index("i")

        barrier = pltpu.get_barrier_semaphore()
        for delta in range(1, N_DEV):
            pl.semaphore_signal(
                barrier, inc=1,
                device_id=((my + delta) % N_DEV,),
                device_id_type=pl.DeviceIdType.MESH,
            )
        pl.semaphore_wait(barrier, N_DEV - 1)

        wq = wq_ref[...]
        wo = wo_ref[...]
        parts = []
        for b in range(B):
            q_b = jnp.dot(x_ref[b], wq, preferred_element_type=jnp.float32)
            ctx = []
            for h in range(H_LOC):
                i = b * H_LOC + h
                q_bh = q_b[:, h * DH:(h + 1) * DH]
                s = lax.dot_general(
                    q_bh, k_ref[i], (((1,), (1,)), ((), ())),
                    preferred_element_type=jnp.float32,
                ) * 0.125
                m = jnp.max(s, axis=-1, keepdims=True)
                w = jnp.exp(s - m)
                w = w / jnp.sum(w, axis=-1, keepdims=True)
                ctx.append(jnp.dot(w, v_ref[i],
                                   preferred_element_type=jnp.float32))
            ctx_b = jnp.concatenate(ctx, axis=1)
            parts.append(jnp.dot(ctx_b, wo,
                                 preferred_element_type=jnp.float32))
        part_ref[...] = jnp.concatenate(parts, axis=0)

        rs = []
        for delta in range(1, N_DEV):
            dst = (my + delta) % N_DEV
            r = pltpu.make_async_remote_copy(
                src_ref=part_ref.at[pl.ds(dst * R8, R8)],
                dst_ref=rs_ref.at[delta],
                send_sem=rs_send.at[delta],
                recv_sem=rs_recv.at[delta],
                device_id=(dst,),
                device_id_type=pl.DeviceIdType.MESH,
            )
            r.start()
            rs.append(r)

        acc = part_ref[pl.ds(my * R8, R8), :]
        for delta in range(1, N_DEV):
            rs[delta - 1].wait_recv()
            acc = acc + rs_ref[delta]
        ag_ref[pl.ds(my * R8, R8), :] = acc

        ag = []
        for delta in range(1, N_DEV):
            dst = (my + delta) % N_DEV
            r = pltpu.make_async_remote_copy(
                src_ref=ag_ref.at[pl.ds(my * R8, R8)],
                dst_ref=ag_ref.at[pl.ds(my * R8, R8)],
                send_sem=ag_send.at[delta],
                recv_sem=ag_recv.at[delta],
                device_id=(dst,),
                device_id_type=pl.DeviceIdType.MESH,
            )
            r.start()
            ag.append(r)
        for delta in range(1, N_DEV):
            src = (my - delta) % N_DEV
            w = pltpu.make_async_remote_copy(
                src_ref=ag_ref.at[pl.ds(my * R8, R8)],
                dst_ref=ag_ref.at[pl.ds(src * R8, R8)],
                send_sem=ag_send.at[delta],
                recv_sem=ag_recv.at[delta],
                device_id=((my + delta) % N_DEV,),
                device_id_type=pl.DeviceIdType.MESH,
            )
            w.wait_recv()

        for r in rs:
            r.wait_send()
        for r in ag:
            r.wait_send()
        out_ref[...] = ag_ref[...].reshape(B, SQ, D_MODEL)

    return pl.pallas_call(
        body,
        out_shape=jax.ShapeDtypeStruct((B, SQ, D_MODEL), jnp.float32),
        in_specs=[pl.BlockSpec(memory_space=pltpu.VMEM)] * 5,
        out_specs=pl.BlockSpec(memory_space=pltpu.VMEM),
        scratch_shapes=[
            pltpu.VMEM((ROWS, D_MODEL), jnp.float32),
            pltpu.VMEM((N_DEV, R8, D_MODEL), jnp.float32),
            pltpu.VMEM((ROWS, D_MODEL), jnp.float32),
            pltpu.SemaphoreType.DMA((N_DEV,)),
            pltpu.SemaphoreType.DMA((N_DEV,)),
            pltpu.SemaphoreType.DMA((N_DEV,)),
            pltpu.SemaphoreType.DMA((N_DEV,)),
        ],
        compiler_params=pltpu.CompilerParams(collective_id=0),
    )(x, Wq, k_loc, v_loc, Wo)
